# baseline (speedup 1.0000x reference)
import functools

import jax
import jax.numpy as jnp
from jax import lax
from jax.experimental import pallas as pl
from jax.experimental.pallas import tpu as pltpu
from jax.experimental.pallas import tpu_sc as plsc

_NUM_CORES = 2
_NUM_SUBCORES = 16
_LANES = 16
_NBUF = 6
_LOOKAHEAD = 5


def _embed_kernel(B, S, E, CW):
    NW = _NUM_CORES * _NUM_SUBCORES
    SP = S // NW
    NH = SP // CW
    NCH = B * NH

    mesh = plsc.VectorSubcoreMesh(core_axis_name="c", subcore_axis_name="s")

    scratch = [
        pltpu.VMEM((B, SP), jnp.int32),
        pltpu.VMEM((SP, E), jnp.float32),
    ]
    NH_SEM = SP // CW
    scratch += [pltpu.VMEM((CW, E), jnp.float32) for _ in range(_NBUF)]
    scratch += [pltpu.SemaphoreType.DMA for _ in range(2 * _NBUF + 1 + NH_SEM)]

    def chunk_bh(ch):
        return ch % B, ch // B

    @functools.partial(
        pl.kernel,
        mesh=mesh,
        out_type=jax.ShapeDtypeStruct((B, S, E), jnp.float32),
        scratch_types=scratch,
    )
    def k(x_hbm, tok_hbm, pos_hbm, out_hbm, idx_v, pos_v, *bufs):
        rows = bufs[0:_NBUF]
        sg = bufs[_NBUF:2 * _NBUF]
        so = bufs[2 * _NBUF:3 * _NBUF]
        s_idx = bufs[3 * _NBUF]
        s_pos = bufs[3 * _NBUF + 1:3 * _NBUF + 1 + NH]

        wid = lax.axis_index("s") * _NUM_CORES + lax.axis_index("c")
        p0 = wid * SP

        for b in range(B):
            pltpu.async_copy(x_hbm.at[b, pl.ds(p0, SP)], idx_v.at[b], s_idx)
        for h in range(NH):
            pltpu.async_copy(
                pos_hbm.at[pl.ds(p0 + h * CW, CW)],
                pos_v.at[pl.ds(h * CW, CW)],
                s_pos[h],
            )
        for b in range(B):
            pltpu.make_async_copy(
                x_hbm.at[b, pl.ds(p0, SP)], idx_v.at[b], s_idx
            ).wait()

        def wait_pos(h):
            pltpu.make_async_copy(
                pos_hbm.at[pl.ds(p0 + h * CW, CW)],
                pos_v.at[pl.ds(h * CW, CW)],
                s_pos[h],
            ).wait()

        def start(ch):
            b, h = chunk_bh(ch)
            g = ch % _NBUF
            pltpu.async_copy(
                tok_hbm.at[idx_v.at[b, pl.ds(h * CW, CW)]], rows[g], sg[g]
            )

        def wait_in(ch):
            b, h = chunk_bh(ch)
            g = ch % _NBUF
            pltpu.make_async_copy(
                tok_hbm.at[idx_v.at[b, pl.ds(h * CW, CW)]], rows[g], sg[g]
            ).wait()

        def out_slice(ch):
            b, h = chunk_bh(ch)
            return out_hbm.at[b, pl.ds(p0 + h * CW, CW)]

        def wait_out(ch):
            g = ch % _NBUF
            pltpu.make_async_copy(rows[g], out_slice(ch), so[g]).wait()

        def add_and_store(ch):
            _, h = chunk_bh(ch)
            g = ch % _NBUF

            @pl.loop(0, CW)
            def _pos(p):
                @plsc.parallel_loop(0, E, step=_LANES, unroll=4)
                def _col(e):
                    pv = pos_v.at[h * CW + p, pl.ds(e, _LANES)][...]
                    plsc.addupdate(rows[g].at[p, pl.ds(e, _LANES)], pv)

            pltpu.async_copy(rows[g], out_slice(ch), so[g])

        for ch in range(min(_LOOKAHEAD, NCH)):
            start(ch)
        for ch in range(NCH):
            if ch % B == 0:
                wait_pos(chunk_bh(ch)[1])
            wait_in(ch)
            add_and_store(ch)
            n = ch + _LOOKAHEAD
            if n < NCH:
                if n - _NBUF >= 0:
                    wait_out(n - _NBUF)
                start(n)
        for ch in range(max(0, NCH - _NBUF), NCH):
            wait_out(ch)

    return k


def kernel(x, tok_table, pos_table):
    B, S = x.shape
    _, E = tok_table.shape
    return _embed_kernel(B, S, E, CW=16)(
        x.astype(jnp.int32), tok_table, pos_table
    )
